# baseline (speedup 1.0000x reference)
import functools

import jax
import jax.numpy as jnp
from jax import lax
from jax.experimental import pallas as pl
from jax.experimental.pallas import tpu as pltpu
from jax.experimental.pallas import tpu_sc as plsc

NC = 2
NS = 16
NW = NC * NS
LANES = 16
BBLK = 256
GROUP = 128
TBLK = 800


def _skew_store_idx(r, h):
    rot = (lax.iota(jnp.int32, LANES) + r) & (LANES - 1)
    return r * 32 + 16 * h + rot


def _skew_load_idx(r0, d):
    i = lax.iota(jnp.int32, LANES)
    rot = (d + r0 + i) & (LANES - 1)
    return (r0 + i) * 32 + 16 * (d // LANES) + rot


def _build_transpose(V, D):
    nblk = V // TBLK
    base_per_w = nblk // NW
    rem = nblk % NW
    ngrp = TBLK // LANES

    mesh = plsc.VectorSubcoreMesh(core_axis_name="c", subcore_axis_name="s")

    @functools.partial(
        pl.kernel,
        out_type=jax.ShapeDtypeStruct((V, D), jnp.float32),
        mesh=mesh,
        compiler_params=pltpu.CompilerParams(
            use_tc_tiling_on_sc=False, needs_layout_passes=False),
        scratch_types=[
            pltpu.VMEM((D, TBLK), jnp.float32),
            pltpu.VMEM((TBLK * D,), jnp.float32),
            pltpu.VMEM((TBLK, D), jnp.float32),
            pltpu.SemaphoreType.DMA,
        ],
    )
    def trans(tok_dm, out_hbm, in_v, skew, tr_v, wsem):
        wid = lax.axis_index("s") * NC + lax.axis_index("c")
        nb_w = base_per_w + jnp.where(wid < rem, 1, 0)
        blk0 = wid * base_per_w + jnp.minimum(wid, rem)

        def body(k, carry):
            c0 = (blk0 + k) * TBLK
            pltpu.sync_copy(tok_dm.at[:, pl.ds(c0, TBLK)], in_v)

            @plsc.parallel_loop(0, ngrp)
            def _(g):
                t0 = g * LANES
                i = lax.iota(jnp.int32, LANES)
                for d in range(D):
                    v = in_v[d, pl.ds(t0, LANES)]
                    rot = (d + t0 + i) & (LANES - 1)
                    idx = (t0 + i) * D + 16 * (d // LANES) + rot
                    plsc.store_scatter(skew, [idx], v)

            @pl.when(k > 0)
            def _():
                pltpu.make_async_copy(tr_v, out_hbm.at[pl.ds(0, TBLK)],
                                      wsem).wait()

            @plsc.parallel_loop(0, TBLK)
            def _(t):
                i = lax.iota(jnp.int32, LANES)
                for h in range(D // LANES):
                    rot = (i + t) & (LANES - 1)
                    idx = t * D + 16 * h + rot
                    tr_v[t, pl.ds(16 * h, LANES)] = plsc.load_gather(
                        skew, [idx])

            pltpu.async_copy(tr_v, out_hbm.at[pl.ds(c0, TBLK)], wsem)
            return carry

        lax.fori_loop(0, nb_w, body, 0)
        pltpu.make_async_copy(tr_v, out_hbm.at[pl.ds(0, TBLK)], wsem).wait()

    return trans


def _build_gather(B, L, D):
    nblk = B // BBLK
    items = L * nblk
    per_w = items // NW
    gpc = BBLK // GROUP
    ngrp = BBLK // LANES

    mesh = plsc.VectorSubcoreMesh(core_axis_name="c", subcore_axis_name="s")

    @functools.partial(
        pl.kernel,
        out_type=jax.ShapeDtypeStruct((L, D, B), jnp.float32),
        mesh=mesh,
        compiler_params=pltpu.CompilerParams(
            use_tc_tiling_on_sc=False, needs_layout_passes=False),
        scratch_types=[
            pltpu.VMEM((BBLK,), jnp.int32),
            pltpu.VMEM((BBLK,), jnp.int32),
            pltpu.VMEM((BBLK, GROUP), jnp.float32),
            pltpu.VMEM((BBLK, GROUP), jnp.float32),
            pltpu.VMEM((BBLK * D,), jnp.float32),
            pltpu.VMEM((D, BBLK), jnp.float32),
            pltpu.VMEM((D, BBLK), jnp.float32),
            pltpu.VMEM((L, D), jnp.float32),
            pltpu.SemaphoreType.DMA,
            pltpu.SemaphoreType.DMA,
            pltpu.SemaphoreType.DMA,
            pltpu.SemaphoreType.DMA,
        ],
    )
    def emb(seq_hbm, tok_hbm, pos_hbm, out_hbm,
            idx_a, idx_b, rows_a, rows_b, skew, tout_a, tout_b, pos_v,
            gsem_a, gsem_b, wsem_a, wsem_b):
        wid = lax.axis_index("s") * NC + lax.axis_index("c")
        item0 = wid * per_w

        pltpu.sync_copy(pos_hbm, pos_v)

        def fire(t, idx_v, rows_v, gsem):
            l = t // nblk
            bb = t % nblk
            pltpu.sync_copy(seq_hbm.at[l, pl.ds(bb * BBLK, BBLK)], idx_v)
            return [
                pltpu.async_copy(tok_hbm.at[idx_v.at[pl.ds(g * GROUP, GROUP)]],
                                 rows_v.at[pl.ds(g * GROUP, GROUP)], gsem)
                for g in range(gpc)
            ]

        def transpose_add(t, rows_v, tout_v):
            l = t // nblk
            pvec = [pos_v[l, pl.ds(h * LANES, LANES)]
                    for h in range(D // LANES)]

            @plsc.parallel_loop(0, BBLK)
            def _(r):
                i = lax.iota(jnp.int32, LANES)
                rot = (i + r) & (LANES - 1)
                for h in range(D // LANES):
                    v = rows_v[r, pl.ds(16 * h, LANES)] + pvec[h]
                    idx = r * D + 16 * h + rot
                    plsc.store_scatter(skew, [idx], v)

            @plsc.parallel_loop(0, ngrp)
            def _(g):
                r0 = g * LANES
                i = lax.iota(jnp.int32, LANES)
                base = (r0 + i) * D
                for d in range(D):
                    rot = (d + r0 + i) & (LANES - 1)
                    idx = base + 16 * (d // LANES) + rot
                    tout_v[d, pl.ds(r0, LANES)] = plsc.load_gather(
                        skew, [idx])

        def out_slice(t):
            l = t // nblk
            bb = t % nblk
            return out_hbm.at[l, :, pl.ds(bb * BBLK, BBLK)]

        def iter_body(i, carry):
            ta = item0 + 2 * i
            tb = item0 + 2 * i + 1
            g_a = fire(ta, idx_a, rows_a, gsem_a)
            g_b = fire(tb, idx_b, rows_b, gsem_b)
            for cp in g_a:
                cp.wait()

            @pl.when(i > 0)
            def _():
                pltpu.make_async_copy(tout_a, out_slice(ta), wsem_a).wait()

            transpose_add(ta, rows_a, tout_a)
            pltpu.async_copy(tout_a, out_slice(ta), wsem_a)
            for cp in g_b:
                cp.wait()

            @pl.when(i > 0)
            def _():
                pltpu.make_async_copy(tout_b, out_slice(tb), wsem_b).wait()

            transpose_add(tb, rows_b, tout_b)
            pltpu.async_copy(tout_b, out_slice(tb), wsem_b)
            return carry

        lax.fori_loop(0, per_w // 2, iter_body, 0)
        pltpu.make_async_copy(tout_a, out_slice(item0), wsem_a).wait()
        pltpu.make_async_copy(tout_b, out_slice(item0), wsem_b).wait()

    return emb


def kernel(seq, token_table, pos_table):
    B, L = seq.shape
    V, D = token_table.shape
    seq_t = jnp.swapaxes(seq, 0, 1).astype(jnp.int32)
    tok_p = jnp.pad(token_table, ((0, 0), (0, GROUP - D)))
    out3 = _build_gather(B, L, D)(seq_t, tok_p, pos_table)
    return jnp.transpose(out3, (2, 0, 1))


# speedup vs baseline: 1.1879x; 1.1879x over previous
import functools

import jax
import jax.numpy as jnp
from jax import lax
from jax.experimental import pallas as pl
from jax.experimental.pallas import tpu as pltpu
from jax.experimental.pallas import tpu_sc as plsc

NC = 2
NS = 16
NW = NC * NS
LANES = 16
BBLK = 512
GROUP = 128
TBLK = 800


def _build_gather(B, L, D):
    nblk = B // BBLK
    items = L * nblk
    per_w = items // NW
    gpc = BBLK // GROUP
    ngrp = BBLK // LANES

    mesh = plsc.VectorSubcoreMesh(core_axis_name="c", subcore_axis_name="s")

    @functools.partial(
        pl.kernel,
        out_type=jax.ShapeDtypeStruct((L, D, B), jnp.float32),
        mesh=mesh,
        compiler_params=pltpu.CompilerParams(
            use_tc_tiling_on_sc=False, needs_layout_passes=False),
        scratch_types=[
            pltpu.VMEM((BBLK,), jnp.int32),
            pltpu.VMEM((BBLK,), jnp.int32),
            pltpu.VMEM((BBLK, D), jnp.float32),
            pltpu.VMEM((BBLK, D), jnp.float32),
            pltpu.VMEM((BBLK * D,), jnp.float32),
            pltpu.VMEM((D, BBLK), jnp.float32),
            pltpu.VMEM((D, BBLK), jnp.float32),
            pltpu.VMEM((L, D), jnp.float32),
            pltpu.SemaphoreType.DMA,
            pltpu.SemaphoreType.DMA,
            pltpu.SemaphoreType.DMA,
            pltpu.SemaphoreType.DMA,
        ],
    )
    def emb(seq_hbm, tok_hbm, pos_hbm, out_hbm,
            idx_a, idx_b, rows_a, rows_b, skew, tout_a, tout_b, pos_v,
            gsem_a, gsem_b, wsem_a, wsem_b):
        wid = lax.axis_index("s") * NC + lax.axis_index("c")
        item0 = wid * per_w

        pltpu.sync_copy(pos_hbm, pos_v)

        def fire(t, idx_v, rows_v, gsem):
            l = t // nblk
            bb = t % nblk
            pltpu.sync_copy(seq_hbm.at[l, pl.ds(bb * BBLK, BBLK)], idx_v)
            return [
                pltpu.async_copy(tok_hbm.at[idx_v.at[pl.ds(g * GROUP, GROUP)]],
                                 rows_v.at[pl.ds(g * GROUP, GROUP)], gsem)
                for g in range(gpc)
            ]

        def transpose_add(t, rows_v, tout_v):
            l = t // nblk
            pvec = [pos_v[l, pl.ds(h * LANES, LANES)]
                    for h in range(D // LANES)]

            @plsc.parallel_loop(0, BBLK)
            def _(r):
                i = lax.iota(jnp.int32, LANES)
                rot = (i + r) & (LANES - 1)
                for h in range(D // LANES):
                    v = rows_v[r, pl.ds(16 * h, LANES)] + pvec[h]
                    idx = r * D + 16 * h + rot
                    plsc.store_scatter(skew, [idx], v)

            @plsc.parallel_loop(0, ngrp)
            def _(g):
                r0 = g * LANES
                i = lax.iota(jnp.int32, LANES)
                base = (r0 + i) * D
                for d in range(D):
                    rot = (d + r0 + i) & (LANES - 1)
                    idx = base + 16 * (d // LANES) + rot
                    tout_v[d, pl.ds(r0, LANES)] = plsc.load_gather(
                        skew, [idx])

        def out_slice(t):
            l = t // nblk
            bb = t % nblk
            return out_hbm.at[l, :, pl.ds(bb * BBLK, BBLK)]

        def drain_gathers(rows_v, gsem):
            pltpu.make_async_copy(rows_v, tok_hbm.at[pl.ds(0, BBLK)],
                                  gsem).wait()

        nit = per_w // 2

        def iter_body(i, carry):
            ta = item0 + 2 * i
            tb = item0 + 2 * i + 1
            drain_gathers(rows_a, gsem_a)

            @pl.when(i > 0)
            def _():
                pltpu.make_async_copy(tout_a, out_slice(ta), wsem_a).wait()

            transpose_add(ta, rows_a, tout_a)

            @pl.when(i + 1 < nit)
            def _():
                fire(ta + 2, idx_a, rows_a, gsem_a)

            pltpu.async_copy(tout_a, out_slice(ta), wsem_a)
            drain_gathers(rows_b, gsem_b)

            @pl.when(i > 0)
            def _():
                pltpu.make_async_copy(tout_b, out_slice(tb), wsem_b).wait()

            transpose_add(tb, rows_b, tout_b)

            @pl.when(i + 1 < nit)
            def _():
                fire(tb + 2, idx_b, rows_b, gsem_b)

            pltpu.async_copy(tout_b, out_slice(tb), wsem_b)
            return carry

        fire(item0, idx_a, rows_a, gsem_a)
        fire(item0 + 1, idx_b, rows_b, gsem_b)
        lax.fori_loop(0, nit, iter_body, 0)
        pltpu.make_async_copy(tout_a, out_slice(item0), wsem_a).wait()
        pltpu.make_async_copy(tout_b, out_slice(item0), wsem_b).wait()

    return emb


def kernel(seq, token_table, pos_table):
    B, L = seq.shape
    V, D = token_table.shape
    seq_t = jnp.swapaxes(seq, 0, 1).astype(jnp.int32)
    out3 = _build_gather(B, L, D)(seq_t, token_table, pos_table)
    return jnp.transpose(out3, (2, 0, 1))


# speedup vs baseline: 1.1893x; 1.0012x over previous
import functools

import jax
import jax.numpy as jnp
from jax import lax
from jax.experimental import pallas as pl
from jax.experimental.pallas import tpu as pltpu
from jax.experimental.pallas import tpu_sc as plsc

NC = 2
NS = 16
NW = NC * NS
LANES = 16
BBLK = 512
GROUP = 128


def _build_gather(B, L, D):
    nblk = B // BBLK
    items = L * nblk
    per_w = items // NW
    gpc = BBLK // GROUP
    ngrp = BBLK // LANES

    mesh = plsc.VectorSubcoreMesh(core_axis_name="c", subcore_axis_name="s")

    @functools.partial(
        pl.kernel,
        out_type=jax.ShapeDtypeStruct((L, D, B), jnp.float32),
        mesh=mesh,
        compiler_params=pltpu.CompilerParams(
            use_tc_tiling_on_sc=False, needs_layout_passes=False),
        scratch_types=[
            pltpu.VMEM((BBLK,), jnp.int32),
            pltpu.VMEM((BBLK,), jnp.int32),
            pltpu.VMEM((BBLK, D), jnp.float32),
            pltpu.VMEM((BBLK, D), jnp.float32),
            pltpu.VMEM((BBLK * D,), jnp.float32),
            pltpu.VMEM((D, BBLK), jnp.float32),
            pltpu.VMEM((D, BBLK), jnp.float32),
            pltpu.VMEM((L, D), jnp.float32),
            pltpu.SemaphoreType.DMA,
            pltpu.SemaphoreType.DMA,
            pltpu.SemaphoreType.DMA,
            pltpu.SemaphoreType.DMA,
        ],
    )
    def emb(seq_hbm, tok_hbm, pos_hbm, out_hbm,
            idx_a, idx_b, rows_a, rows_b, skew, tout_a, tout_b, pos_v,
            gsem_a, gsem_b, wsem_a, wsem_b):
        wid = lax.axis_index("s") * NC + lax.axis_index("c")
        item0 = wid * per_w

        pltpu.sync_copy(pos_hbm, pos_v)

        def fire(t, idx_v, rows_v, gsem):
            l = t // nblk
            bb = t % nblk
            pltpu.sync_copy(seq_hbm.at[l, pl.ds(bb * BBLK, BBLK)], idx_v)
            return [
                pltpu.async_copy(tok_hbm.at[idx_v.at[pl.ds(g * GROUP, GROUP)]],
                                 rows_v.at[pl.ds(g * GROUP, GROUP)], gsem)
                for g in range(gpc)
            ]

        def transpose_add(t, rows_v, tout_v):
            l = t // nblk
            pvec = [pos_v[l, pl.ds(h * LANES, LANES)]
                    for h in range(D // LANES)]

            @plsc.parallel_loop(0, BBLK)
            def _(r):
                i = lax.iota(jnp.int32, LANES)
                rot = (i + r) & (LANES - 1)
                for h in range(D // LANES):
                    v = rows_v[r, pl.ds(16 * h, LANES)] + pvec[h]
                    idx = r * D + 16 * h + rot
                    plsc.store_scatter(skew, [idx], v)

            @plsc.parallel_loop(0, ngrp)
            def _(g):
                r0 = g * LANES
                i = lax.iota(jnp.int32, LANES)
                base = (r0 + i) * D
                for d in range(D):
                    rot = (d + r0 + i) & (LANES - 1)
                    idx = base + 16 * (d // LANES) + rot
                    tout_v[d, pl.ds(r0, LANES)] = plsc.load_gather(
                        skew, [idx])

        def out_slice(t):
            l = t // nblk
            bb = t % nblk
            return out_hbm.at[l, :, pl.ds(bb * BBLK, BBLK)]

        def drain_gathers(rows_v, gsem):
            pltpu.make_async_copy(rows_v, tok_hbm.at[pl.ds(0, BBLK)],
                                  gsem).wait()

        nit = per_w // 2

        def iter_body(i, carry):
            ta = item0 + 2 * i
            tb = item0 + 2 * i + 1
            drain_gathers(rows_a, gsem_a)

            @pl.when(i > 0)
            def _():
                pltpu.make_async_copy(tout_a, out_slice(ta), wsem_a).wait()

            transpose_add(ta, rows_a, tout_a)

            @pl.when(i + 1 < nit)
            def _():
                fire(ta + 2, idx_a, rows_a, gsem_a)

            pltpu.async_copy(tout_a, out_slice(ta), wsem_a)
            drain_gathers(rows_b, gsem_b)

            @pl.when(i > 0)
            def _():
                pltpu.make_async_copy(tout_b, out_slice(tb), wsem_b).wait()

            transpose_add(tb, rows_b, tout_b)

            @pl.when(i + 1 < nit)
            def _():
                fire(tb + 2, idx_b, rows_b, gsem_b)

            pltpu.async_copy(tout_b, out_slice(tb), wsem_b)
            return carry

        fire(item0, idx_a, rows_a, gsem_a)
        fire(item0 + 1, idx_b, rows_b, gsem_b)
        lax.fori_loop(0, nit, iter_body, 0)
        pltpu.make_async_copy(tout_a, out_slice(item0), wsem_a).wait()
        pltpu.make_async_copy(tout_b, out_slice(item0), wsem_b).wait()

    return emb


def kernel(seq, token_table, pos_table):
    B, L = seq.shape
    V, D = token_table.shape
    seq_t = jnp.swapaxes(seq, 0, 1).astype(jnp.int32)
    out3 = _build_gather(B, L, D)(seq_t, token_table, pos_table)
    return jnp.transpose(out3, (2, 0, 1))
